# Initial kernel scaffold; baseline (speedup 1.0000x reference)
import jax
import jax.numpy as jnp
from jax.experimental import pallas as pl


def kernel(x, edge_index, edge_attr, u, batch, polar_pos, eW1, eb1, eW2, eb2, nW1, nb1, nW2, nb2, gW1, gb1, gW2, gb2):
    raise NotImplementedError("write your pallas kernel here")


# speedup vs baseline: 3.9628x; 3.9628x over previous
import functools

import jax
import jax.numpy as jnp
from jax import lax
from jax.experimental import pallas as pl
from jax.experimental.pallas import tpu as pltpu
from jax.experimental.pallas import tpu_sc as plsc

_N = 10000
_E = 320000
_D = 128
_DE = 16
_G = 16
_DU = 32
_H = 128
_STEPS = 2

_F32 = jnp.float32
_HIGH = lax.Precision.HIGHEST

_TILE_N = 1000
_TILE_E = 2000

_NC = 2
_NS = 16
_NW = _NC * _NS
_EPW = _E // _NW
_CH = 128
_NCH = -(-_EPW // _CH)
_LAST = _EPW - _CH
_FULL = _EPW // _CH
_TAIL = _EPW - _FULL * _CH
_NCP = 10
_NPS = _N // _NCP
_NCH2 = 81
_NT = 26


def _oh16(b_block):
    iota = lax.broadcasted_iota(jnp.int32, (1, _G), 1).astype(_F32)
    return (b_block == iota).astype(_F32)


def _stats_body(b_r, p_r, cnt_r, ps_r):
    i = pl.program_id(0)
    oh = _oh16(b_r[...])

    @pl.when(i == 0)
    def _():
        cnt_r[...] = jnp.zeros_like(cnt_r)
        ps_r[...] = jnp.zeros_like(ps_r)

    ones = jnp.ones((_TILE_N, _H), _F32)
    cnt_r[...] += lax.dot_general(oh, ones, (((0,), (0,)), ((), ())),
                                  precision=_HIGH)
    ps_r[...] += lax.dot_general(oh, p_r[...], (((0,), (0,)), ((), ())),
                                 precision=_HIGH)


def _stats(batchf, polar):
    return pl.pallas_call(
        _stats_body,
        grid=(_N // _TILE_N,),
        in_specs=[
            pl.BlockSpec((_TILE_N, 1), lambda i: (i, 0)),
            pl.BlockSpec((_TILE_N, 2), lambda i: (i, 0)),
        ],
        out_specs=[
            pl.BlockSpec((_G, _H), lambda i: (0, 0)),
            pl.BlockSpec((_G, 2), lambda i: (0, 0)),
        ],
        out_shape=[
            jax.ShapeDtypeStruct((_G, _H), _F32),
            jax.ShapeDtypeStruct((_G, 2), _F32),
        ],
    )(batchf, polar)


def _prep_body(x_r, b_r, u_r, ws_r, wd_r, wu_r, eb1_r, ps_r, pd_r):
    x = x_r[...]
    oh = _oh16(b_r[...])
    uw = jnp.dot(u_r[...], wu_r[...], precision=_HIGH)
    ps_r[...] = (jnp.dot(x, ws_r[...], precision=_HIGH)
                 + jnp.dot(oh, uw, precision=_HIGH) + eb1_r[...])
    pd_r[...] = jnp.dot(x, wd_r[...], precision=_HIGH)


def _prep(x, batchf, u, ws, wd, wu, eb1):
    return pl.pallas_call(
        _prep_body,
        grid=(_N // _TILE_N,),
        in_specs=[
            pl.BlockSpec((_TILE_N, _D), lambda i: (i, 0)),
            pl.BlockSpec((_TILE_N, 1), lambda i: (i, 0)),
            pl.BlockSpec((_G, _DU), lambda i: (0, 0)),
            pl.BlockSpec((_D, _H), lambda i: (0, 0)),
            pl.BlockSpec((_D, _H), lambda i: (0, 0)),
            pl.BlockSpec((_DU, _H), lambda i: (0, 0)),
            pl.BlockSpec((1, _H), lambda i: (0, 0)),
        ],
        out_specs=[pl.BlockSpec((_TILE_N, _H), lambda i: (i, 0))] * 2,
        out_shape=[jax.ShapeDtypeStruct((_N, _H), _F32)] * 2,
    )(x, batchf, u, ws, wd, wu, eb1)


def _edge_body(hsum_r, ea_r, we_r, w2_r, eb2_r, out_r):
    h = jnp.maximum(
        hsum_r[...] + jnp.dot(ea_r[...], we_r[...], precision=_HIGH), 0.0)
    out_r[...] = jnp.dot(h, w2_r[...], precision=_HIGH) + eb2_r[...]


def _edge(hsum, ea, we, w2, eb2):
    return pl.pallas_call(
        _edge_body,
        grid=(_E // _TILE_E,),
        in_specs=[
            pl.BlockSpec((_TILE_E, _H), lambda i: (i, 0)),
            pl.BlockSpec((_TILE_E, _DE), lambda i: (i, 0)),
            pl.BlockSpec((_DE, _H), lambda i: (0, 0)),
            pl.BlockSpec((_H, _DE), lambda i: (0, 0)),
            pl.BlockSpec((1, _DE), lambda i: (0, 0)),
        ],
        out_specs=pl.BlockSpec((_TILE_E, _DE), lambda i: (i, 0)),
        out_shape=jax.ShapeDtypeStruct((_E, _DE), _F32),
    )(hsum, ea, we, w2, eb2)


def _node_body(x_r, a0_r, a1_r, b_r, u_r, wx_r, wa_r, wu_r, nb1_r, w2_r,
               nb2_r, xn_r, xs_r):
    i = pl.program_id(0)
    x = x_r[...]
    oh = _oh16(b_r[...])
    uw = jnp.dot(u_r[...], wu_r[...], precision=_HIGH)
    agg = a0_r[...] + a1_r[...]
    npre = (jnp.dot(x, wx_r[...], precision=_HIGH)
            + jnp.dot(agg, wa_r[...], precision=_HIGH)
            + jnp.dot(oh, uw, precision=_HIGH) + nb1_r[...])
    xn = jnp.dot(jnp.maximum(npre, 0.0), w2_r[...], precision=_HIGH) + nb2_r[...]
    xn_r[...] = xn

    @pl.when(i == 0)
    def _():
        xs_r[...] = jnp.zeros_like(xs_r)

    xs_r[...] += lax.dot_general(oh, xn, (((0,), (0,)), ((), ())),
                                 precision=_HIGH)


def _node(x, a0, a1, batchf, u, wx, wa, wu, nb1, w2, nb2):
    return pl.pallas_call(
        _node_body,
        grid=(_N // _TILE_N,),
        in_specs=[
            pl.BlockSpec((_TILE_N, _D), lambda i: (i, 0)),
            pl.BlockSpec((_TILE_N, _DE), lambda i: (i, 0)),
            pl.BlockSpec((_TILE_N, _DE), lambda i: (i, 0)),
            pl.BlockSpec((_TILE_N, 1), lambda i: (i, 0)),
            pl.BlockSpec((_G, _DU), lambda i: (0, 0)),
            pl.BlockSpec((_D, _H), lambda i: (0, 0)),
            pl.BlockSpec((_DE, _H), lambda i: (0, 0)),
            pl.BlockSpec((_DU, _H), lambda i: (0, 0)),
            pl.BlockSpec((1, _H), lambda i: (0, 0)),
            pl.BlockSpec((_H, _D), lambda i: (0, 0)),
            pl.BlockSpec((1, _D), lambda i: (0, 0)),
        ],
        out_specs=[
            pl.BlockSpec((_TILE_N, _D), lambda i: (i, 0)),
            pl.BlockSpec((_G, _H), lambda i: (0, 0)),
        ],
        out_shape=[
            jax.ShapeDtypeStruct((_N, _D), _F32),
            jax.ShapeDtypeStruct((_G, _H), _F32),
        ],
    )(x, a0, a1, batchf, u, wx, wa, wu, nb1, w2, nb2)


def _glob_body(u_r, xs_r, ps_r, cnt_r, gwu_r, gwx_r, gwp_r, gb1_r, gw2_r,
               gb2_r, un_r):
    c = jnp.maximum(cnt_r[...], 1.0)
    xm = xs_r[...] / c
    pm = ps_r[...] / c[:, :2]
    gpre = (jnp.dot(u_r[...], gwu_r[...], precision=_HIGH)
            + jnp.dot(xm, gwx_r[...], precision=_HIGH)
            + jnp.dot(pm, gwp_r[...], precision=_HIGH) + gb1_r[...])
    un_r[...] = jnp.dot(jnp.maximum(gpre, 0.0), gw2_r[...],
                        precision=_HIGH) + gb2_r[...]


def _glob(u, xsum, psum, cnt, gwu, gwx, gwp, gb1, gw2, gb2):
    return pl.pallas_call(
        _glob_body,
        out_shape=jax.ShapeDtypeStruct((_G, _DU), _F32),
    )(u, xsum, psum, cnt, gwu, gwx, gwp, gb1, gw2, gb2)


def _gather_sc(ps, pd, srcp, dstp):
    mesh = plsc.VectorSubcoreMesh(core_axis_name="c", subcore_axis_name="s")

    @functools.partial(
        pl.kernel,
        mesh=mesh,
        out_type=jax.ShapeDtypeStruct((_E, _H), _F32),
        scratch_types=[
            pltpu.VMEM((_EPW,), jnp.int32),
            pltpu.VMEM((_EPW,), jnp.int32),
        ] + [pltpu.VMEM((_CH, _H), _F32)] * 6
          + [pltpu.SemaphoreType.DMA] * 9,
        compiler_params=pltpu.CompilerParams(use_tc_tiling_on_sc=False),
    )
    def k(ps_h, pd_h, src_h, dst_h, hsum_h, src_v, dst_v,
          rs0, rd0, rs1, rd1, rs2, rd2,
          gs0, gd0, w0, gs1, gd1, w1, gs2, gd2, w2):
        wid = lax.axis_index("s") * _NC + lax.axis_index("c")
        base = wid * _EPW
        pltpu.sync_copy(src_h.at[pl.ds(base, _EPW)], src_v)
        pltpu.sync_copy(dst_h.at[pl.ds(base, _EPW)], dst_v)

        rs = (rs0, rs1, rs2)
        rd = (rd0, rd1, rd2)
        gs = (gs0, gs1, gs2)
        gd = (gd0, gd1, gd2)
        ws = (w0, w1, w2)

        def s0_of(i):
            return jnp.minimum(i * _CH, _LAST)

        def g_start(i, sl):
            s0 = s0_of(i)
            pltpu.make_async_copy(
                ps_h.at[src_v.at[pl.ds(s0, _CH)]], rs[sl], gs[sl]).start()
            pltpu.make_async_copy(
                pd_h.at[dst_v.at[pl.ds(s0, _CH)]], rd[sl], gd[sl]).start()

        def g_wait(i, sl):
            s0 = s0_of(i)
            pltpu.make_async_copy(
                ps_h.at[src_v.at[pl.ds(s0, _CH)]], rs[sl], gs[sl]).wait()
            pltpu.make_async_copy(
                pd_h.at[dst_v.at[pl.ds(s0, _CH)]], rd[sl], gd[sl]).wait()

        def compute(sl):
            a, b = rs[sl], rd[sl]

            def cb(r, carry):
                for c in range(_H // 16):
                    s = pl.ds(c * 16, 16)
                    a[r, s] = a[r, s] + b[r, s]
                return carry

            lax.fori_loop(0, _CH, cb, 0)

        def w_start(i, sl):
            s0 = s0_of(i)
            pltpu.make_async_copy(
                rs[sl], hsum_h.at[pl.ds(base + s0, _CH)], ws[sl]).start()

        def w_wait(i, sl):
            s0 = s0_of(i)
            pltpu.make_async_copy(
                rs[sl], hsum_h.at[pl.ds(base + s0, _CH)], ws[sl]).wait()

        g_start(0, 0)
        g_start(1, 1)
        g_wait(0, 0)
        compute(0)
        w_start(0, 0)

        def body(t, carry):
            i0 = 3 * t

            @pl.when(t > 0)
            def _():
                w_wait(i0 - 1, 2)

            g_start(i0 + 2, 2)
            g_wait(i0 + 1, 1)
            compute(1)
            w_start(i0 + 1, 1)

            w_wait(i0, 0)
            g_start(i0 + 3, 0)
            g_wait(i0 + 2, 2)
            compute(2)
            w_start(i0 + 2, 2)

            w_wait(i0 + 1, 1)
            g_start(i0 + 4, 1)
            g_wait(i0 + 3, 0)
            compute(0)
            w_start(i0 + 3, 0)
            return carry

        lax.fori_loop(0, _NT, body, 0)

        w_wait(_NCH2 - 4, 2)
        g_start(_NCH2 - 1, 2)
        g_wait(_NCH2 - 2, 1)
        compute(1)
        w_start(_NCH2 - 2, 1)
        g_wait(_NCH2 - 1, 2)
        compute(2)
        w_start(_NCH2 - 1, 2)
        w_wait(_NCH2 - 3, 0)
        w_wait(_NCH2 - 2, 1)
        w_wait(_NCH2 - 1, 2)

    return k(ps, pd, srcp, dstp)


def _scatter_sc(ea, dstp):
    mesh = plsc.VectorSubcoreMesh(core_axis_name="c", subcore_axis_name="s")

    @functools.partial(
        pl.kernel,
        mesh=mesh,
        out_type=jax.ShapeDtypeStruct((_NC, _N, _DE), _F32),
        scratch_types=[
            pltpu.VMEM((_CH,), jnp.int32),
            pltpu.VMEM((_CH, _DE), _F32),
            pltpu.VMEM((_TAIL,), jnp.int32),
            pltpu.VMEM((_TAIL, _DE), _F32),
            pltpu.VMEM((_NPS, _DE), _F32),
            pltpu.VMEM_SHARED((_N, _DE), _F32),
        ],
        compiler_params=pltpu.CompilerParams(use_tc_tiling_on_sc=False),
    )
    def k(ea_h, dst_h, out_h, idx_v, val_v, idxt_v, valt_v, z_v, agg_sh):
        cid = lax.axis_index("c")
        sid = lax.axis_index("s")
        wid = sid * _NC + cid
        base = wid * _EPW

        @pl.when(sid < _NCP)
        def _():
            def zb(i, carry):
                z_v[i, :] = jnp.zeros((_DE,), _F32)
                return carry

            lax.fori_loop(0, _NPS, zb, 0)
            pltpu.sync_copy(z_v, agg_sh.at[pl.ds(sid * _NPS, _NPS)])

        plsc.subcore_barrier()

        def body(i, carry):
            s0 = base + i * _CH
            pltpu.sync_copy(dst_h.at[pl.ds(s0, _CH)], idx_v)
            pltpu.sync_copy(ea_h.at[pl.ds(s0, _CH)], val_v)
            pltpu.sync_copy(val_v, agg_sh.at[idx_v], add=True)
            return carry

        lax.fori_loop(0, _FULL, body, 0)
        s0 = base + _FULL * _CH
        pltpu.sync_copy(dst_h.at[pl.ds(s0, _TAIL)], idxt_v)
        pltpu.sync_copy(ea_h.at[pl.ds(s0, _TAIL)], valt_v)
        pltpu.sync_copy(valt_v, agg_sh.at[idxt_v], add=True)
        plsc.subcore_barrier()

        @pl.when(sid < _NCP)
        def _():
            pltpu.sync_copy(agg_sh.at[pl.ds(sid * _NPS, _NPS)],
                            out_h.at[cid, pl.ds(sid * _NPS, _NPS)])

    return k(ea, dstp)


def kernel(x, edge_index, edge_attr, u, batch, polar_pos,
           eW1, eb1, eW2, eb2, nW1, nb1, nW2, nb2, gW1, gb1, gW2, gb2):
    src = edge_index[0]
    dst = edge_index[1]
    batchf = batch.astype(_F32).reshape(_N, 1)

    eWs, eWd = eW1[:_D], eW1[_D:2 * _D]
    eWe, eWu = eW1[2 * _D:2 * _D + _DE], eW1[2 * _D + _DE:]
    nWx, nWa, nWu = nW1[:_D], nW1[_D:_D + _DE], nW1[_D + _DE:]
    gWu, gWx, gWp = gW1[:_DU], gW1[_DU:_DU + _D], gW1[_DU + _D:]
    eb1r, eb2r = eb1.reshape(1, _H), eb2.reshape(1, _DE)
    nb1r, nb2r = nb1.reshape(1, _H), nb2.reshape(1, _D)
    gb1r, gb2r = gb1.reshape(1, _H), gb2.reshape(1, _DU)

    cnt, psum = _stats(batchf, polar_pos)

    x_cat, g_cat = [x], [u]
    ea = edge_attr
    for _ in range(_STEPS):
        ps, pd = _prep(x, batchf, u, eWs, eWd, eWu, eb1r)
        hsum = _gather_sc(ps, pd, src, dst)
        ea = _edge(hsum, ea, eWe, eW2, eb2r)
        aggp = _scatter_sc(ea, dst)
        x, xsum = _node(x, aggp[0], aggp[1], batchf, u, nWx, nWa, nWu,
                        nb1r, nW2, nb2r)
        u = _glob(u, xsum, psum, cnt, gWu, gWx, gWp, gb1r, gW2, gb2r)
        x_cat.append(x)
        g_cat.append(u)
    return jnp.concatenate(x_cat, axis=1), jnp.concatenate(g_cat, axis=1)


# speedup vs baseline: 3.9695x; 1.0017x over previous
import functools

import jax
import jax.numpy as jnp
from jax import lax
from jax.experimental import pallas as pl
from jax.experimental.pallas import tpu as pltpu
from jax.experimental.pallas import tpu_sc as plsc

_N = 10000
_E = 320000
_D = 128
_DE = 16
_G = 16
_DU = 32
_H = 128
_STEPS = 2

_F32 = jnp.float32
_HIGH = lax.Precision.HIGHEST

_TILE_N = 1000
_TILE_E = 6400

_NC = 2
_NS = 16
_NW = _NC * _NS
_EPW = _E // _NW
_CH = 128
_NCH = -(-_EPW // _CH)
_LAST = _EPW - _CH
_FULL = _EPW // _CH
_TAIL = _EPW - _FULL * _CH
_NCP = 10
_NPS = _N // _NCP
_NCH2 = 81
_NT = 26


def _oh16(b_block):
    iota = lax.broadcasted_iota(jnp.int32, (1, _G), 1).astype(_F32)
    return (b_block == iota).astype(_F32)


def _stats_body(b_r, p_r, cnt_r, ps_r):
    i = pl.program_id(0)
    oh = _oh16(b_r[...])

    @pl.when(i == 0)
    def _():
        cnt_r[...] = jnp.zeros_like(cnt_r)
        ps_r[...] = jnp.zeros_like(ps_r)

    ones = jnp.ones((_TILE_N, _H), _F32)
    cnt_r[...] += lax.dot_general(oh, ones, (((0,), (0,)), ((), ())),
                                  precision=_HIGH)
    ps_r[...] += lax.dot_general(oh, p_r[...], (((0,), (0,)), ((), ())),
                                 precision=_HIGH)


def _stats(batchf, polar):
    return pl.pallas_call(
        _stats_body,
        grid=(_N // _TILE_N,),
        in_specs=[
            pl.BlockSpec((_TILE_N, 1), lambda i: (i, 0)),
            pl.BlockSpec((_TILE_N, 2), lambda i: (i, 0)),
        ],
        out_specs=[
            pl.BlockSpec((_G, _H), lambda i: (0, 0)),
            pl.BlockSpec((_G, 2), lambda i: (0, 0)),
        ],
        out_shape=[
            jax.ShapeDtypeStruct((_G, _H), _F32),
            jax.ShapeDtypeStruct((_G, 2), _F32),
        ],
    )(batchf, polar)


def _prep_body(x_r, b_r, u_r, ws_r, wd_r, wu_r, eb1_r, ps_r, pd_r):
    x = x_r[...]
    oh = _oh16(b_r[...])
    uw = jnp.dot(u_r[...], wu_r[...], precision=_HIGH)
    ps_r[...] = (jnp.dot(x, ws_r[...], precision=_HIGH)
                 + jnp.dot(oh, uw, precision=_HIGH) + eb1_r[...])
    pd_r[...] = jnp.dot(x, wd_r[...], precision=_HIGH)


def _prep(x, batchf, u, ws, wd, wu, eb1):
    return pl.pallas_call(
        _prep_body,
        grid=(_N // _TILE_N,),
        in_specs=[
            pl.BlockSpec((_TILE_N, _D), lambda i: (i, 0)),
            pl.BlockSpec((_TILE_N, 1), lambda i: (i, 0)),
            pl.BlockSpec((_G, _DU), lambda i: (0, 0)),
            pl.BlockSpec((_D, _H), lambda i: (0, 0)),
            pl.BlockSpec((_D, _H), lambda i: (0, 0)),
            pl.BlockSpec((_DU, _H), lambda i: (0, 0)),
            pl.BlockSpec((1, _H), lambda i: (0, 0)),
        ],
        out_specs=[pl.BlockSpec((_TILE_N, _H), lambda i: (i, 0))] * 2,
        out_shape=[jax.ShapeDtypeStruct((_N, _H), _F32)] * 2,
    )(x, batchf, u, ws, wd, wu, eb1)


def _edge_body(hsum_r, ea_r, we_r, w2_r, eb2_r, out_r):
    eaw = jnp.dot(ea_r[...], we_r[...], precision=_HIGH)
    h = jnp.maximum(hsum_r[...] + eaw, 0.0)
    out_r[...] = jnp.dot(h, w2_r[...], precision=_HIGH) + eb2_r[...]


def _edge(hsum, ea, we, w2, eb2r):
    return pl.pallas_call(
        _edge_body,
        grid=(_E // _TILE_E,),
        in_specs=[
            pl.BlockSpec((_TILE_E, _H), lambda i: (i, 0)),
            pl.BlockSpec((_TILE_E, _DE), lambda i: (i, 0)),
            pl.BlockSpec((_DE, _H), lambda i: (0, 0)),
            pl.BlockSpec((_H, _DE), lambda i: (0, 0)),
            pl.BlockSpec((1, _DE), lambda i: (0, 0)),
        ],
        out_specs=pl.BlockSpec((_TILE_E, _DE), lambda i: (i, 0)),
        out_shape=jax.ShapeDtypeStruct((_E, _DE), _F32),
    )(hsum, ea, we, w2, eb2r)


def _node_body(x_r, a0_r, a1_r, b_r, u_r, wx_r, wa_r, wu_r, nb1_r, w2_r,
               nb2_r, xn_r, xs_r):
    i = pl.program_id(0)
    x = x_r[...]
    oh = _oh16(b_r[...])
    uw = jnp.dot(u_r[...], wu_r[...], precision=_HIGH)
    agg = a0_r[...] + a1_r[...]
    npre = (jnp.dot(x, wx_r[...], precision=_HIGH)
            + jnp.dot(agg, wa_r[...], precision=_HIGH)
            + jnp.dot(oh, uw, precision=_HIGH) + nb1_r[...])
    xn = jnp.dot(jnp.maximum(npre, 0.0), w2_r[...], precision=_HIGH) + nb2_r[...]
    xn_r[...] = xn

    @pl.when(i == 0)
    def _():
        xs_r[...] = jnp.zeros_like(xs_r)

    xs_r[...] += lax.dot_general(oh, xn, (((0,), (0,)), ((), ())),
                                 precision=_HIGH)


def _node(x, a0, a1, batchf, u, wx, wa, wu, nb1, w2, nb2):
    return pl.pallas_call(
        _node_body,
        grid=(_N // _TILE_N,),
        in_specs=[
            pl.BlockSpec((_TILE_N, _D), lambda i: (i, 0)),
            pl.BlockSpec((_TILE_N, _DE), lambda i: (i, 0)),
            pl.BlockSpec((_TILE_N, _DE), lambda i: (i, 0)),
            pl.BlockSpec((_TILE_N, 1), lambda i: (i, 0)),
            pl.BlockSpec((_G, _DU), lambda i: (0, 0)),
            pl.BlockSpec((_D, _H), lambda i: (0, 0)),
            pl.BlockSpec((_DE, _H), lambda i: (0, 0)),
            pl.BlockSpec((_DU, _H), lambda i: (0, 0)),
            pl.BlockSpec((1, _H), lambda i: (0, 0)),
            pl.BlockSpec((_H, _D), lambda i: (0, 0)),
            pl.BlockSpec((1, _D), lambda i: (0, 0)),
        ],
        out_specs=[
            pl.BlockSpec((_TILE_N, _D), lambda i: (i, 0)),
            pl.BlockSpec((_G, _H), lambda i: (0, 0)),
        ],
        out_shape=[
            jax.ShapeDtypeStruct((_N, _D), _F32),
            jax.ShapeDtypeStruct((_G, _H), _F32),
        ],
    )(x, a0, a1, batchf, u, wx, wa, wu, nb1, w2, nb2)


def _glob_body(u_r, xs_r, ps_r, cnt_r, gwu_r, gwx_r, gwp_r, gb1_r, gw2_r,
               gb2_r, un_r):
    c = jnp.maximum(cnt_r[...], 1.0)
    xm = xs_r[...] / c
    pm = ps_r[...] / c[:, :2]
    gpre = (jnp.dot(u_r[...], gwu_r[...], precision=_HIGH)
            + jnp.dot(xm, gwx_r[...], precision=_HIGH)
            + jnp.dot(pm, gwp_r[...], precision=_HIGH) + gb1_r[...])
    un_r[...] = jnp.dot(jnp.maximum(gpre, 0.0), gw2_r[...],
                        precision=_HIGH) + gb2_r[...]


def _glob(u, xsum, psum, cnt, gwu, gwx, gwp, gb1, gw2, gb2):
    return pl.pallas_call(
        _glob_body,
        out_shape=jax.ShapeDtypeStruct((_G, _DU), _F32),
    )(u, xsum, psum, cnt, gwu, gwx, gwp, gb1, gw2, gb2)


def _gather_sc(ps, pd, srcp, dstp):
    mesh = plsc.VectorSubcoreMesh(core_axis_name="c", subcore_axis_name="s")

    @functools.partial(
        pl.kernel,
        mesh=mesh,
        out_type=jax.ShapeDtypeStruct((_E, _H), _F32),
        scratch_types=[
            pltpu.VMEM((_EPW,), jnp.int32),
            pltpu.VMEM((_EPW,), jnp.int32),
        ] + [pltpu.VMEM((_CH, _H), _F32)] * 6
          + [pltpu.SemaphoreType.DMA] * 9,
        compiler_params=pltpu.CompilerParams(use_tc_tiling_on_sc=False),
    )
    def k(ps_h, pd_h, src_h, dst_h, hsum_h, src_v, dst_v,
          rs0, rd0, rs1, rd1, rs2, rd2,
          gs0, gd0, w0, gs1, gd1, w1, gs2, gd2, w2):
        wid = lax.axis_index("s") * _NC + lax.axis_index("c")
        base = wid * _EPW
        pltpu.sync_copy(src_h.at[pl.ds(base, _EPW)], src_v)
        pltpu.sync_copy(dst_h.at[pl.ds(base, _EPW)], dst_v)

        rs = (rs0, rs1, rs2)
        rd = (rd0, rd1, rd2)
        gs = (gs0, gs1, gs2)
        gd = (gd0, gd1, gd2)
        ws = (w0, w1, w2)

        def s0_of(i):
            return jnp.minimum(i * _CH, _LAST)

        def g_start(i, sl):
            s0 = s0_of(i)
            pltpu.make_async_copy(
                ps_h.at[src_v.at[pl.ds(s0, _CH)]], rs[sl], gs[sl]).start()
            pltpu.make_async_copy(
                pd_h.at[dst_v.at[pl.ds(s0, _CH)]], rd[sl], gd[sl]).start()

        def g_wait(i, sl):
            s0 = s0_of(i)
            pltpu.make_async_copy(
                ps_h.at[src_v.at[pl.ds(s0, _CH)]], rs[sl], gs[sl]).wait()
            pltpu.make_async_copy(
                pd_h.at[dst_v.at[pl.ds(s0, _CH)]], rd[sl], gd[sl]).wait()

        def compute(sl):
            a, b = rs[sl], rd[sl]

            def cb(r, carry):
                for c in range(_H // 16):
                    s = pl.ds(c * 16, 16)
                    a[r, s] = a[r, s] + b[r, s]
                return carry

            lax.fori_loop(0, _CH, cb, 0)

        def w_start(i, sl):
            s0 = s0_of(i)
            pltpu.make_async_copy(
                rs[sl], hsum_h.at[pl.ds(base + s0, _CH)], ws[sl]).start()

        def w_wait(i, sl):
            s0 = s0_of(i)
            pltpu.make_async_copy(
                rs[sl], hsum_h.at[pl.ds(base + s0, _CH)], ws[sl]).wait()

        def chunk(i, carry):
            g_start(i, 0)
            g_wait(i, 0)
            compute(0)
            w_start(i, 0)
            w_wait(i, 0)
            return carry

        lax.fori_loop(0, _NCH, chunk, 0)

    return k(ps, pd, srcp, dstp)


def _scatter_sc(ea, dstp):
    mesh = plsc.VectorSubcoreMesh(core_axis_name="c", subcore_axis_name="s")

    @functools.partial(
        pl.kernel,
        mesh=mesh,
        out_type=jax.ShapeDtypeStruct((_NC, _N, _DE), _F32),
        scratch_types=(
            [pltpu.VMEM((_CH,), jnp.int32)] * 4
            + [pltpu.VMEM((_CH, _DE), _F32)] * 4
            + [
                pltpu.VMEM((_TAIL,), jnp.int32),
                pltpu.VMEM((_TAIL, _DE), _F32),
                pltpu.VMEM((_NPS, _DE), _F32),
                pltpu.VMEM_SHARED((_N, _DE), _F32),
            ]
            + [pltpu.SemaphoreType.DMA] * 8
        ),
        compiler_params=pltpu.CompilerParams(use_tc_tiling_on_sc=False),
    )
    def k(ea_h, dst_h, out_h, idx0, idx1, idx2, idx3,
          val0, val1, val2, val3, idxt_v, valt_v, z_v, agg_sh,
          cs0, cs1, cs2, cs3, ss0, ss1, ss2, ss3):
        cid = lax.axis_index("c")
        sid = lax.axis_index("s")
        wid = sid * _NC + cid
        base = wid * _EPW
        idxb = (idx0, idx1, idx2, idx3)
        val = (val0, val1, val2, val3)
        cs = (cs0, cs1, cs2, cs3)
        ss = (ss0, ss1, ss2, ss3)

        @pl.when(sid < _NCP)
        def _():
            def zb(i, carry):
                z_v[i, :] = jnp.zeros((_DE,), _F32)
                return carry

            lax.fori_loop(0, _NPS, zb, 0)
            pltpu.sync_copy(z_v, agg_sh.at[pl.ds(sid * _NPS, _NPS)])

        plsc.subcore_barrier()

        def l_start(i, sl):
            s0 = base + i * _CH
            pltpu.make_async_copy(
                dst_h.at[pl.ds(s0, _CH)], idxb[sl], cs[sl]).start()
            pltpu.make_async_copy(
                ea_h.at[pl.ds(s0, _CH), :], val[sl], cs[sl]).start()

        def l_wait(i, sl):
            s0 = base + i * _CH
            pltpu.make_async_copy(
                dst_h.at[pl.ds(s0, _CH)], idxb[sl], cs[sl]).wait()
            pltpu.make_async_copy(
                ea_h.at[pl.ds(s0, _CH), :], val[sl], cs[sl]).wait()

        def s_start(sl):
            pltpu.async_copy(val[sl], agg_sh.at[idxb[sl]], ss[sl], add=True)

        def s_wait(sl):
            pltpu.make_async_copy(val[sl], agg_sh.at[idxb[sl]], ss[sl]).wait()

        l_start(0, 0)
        l_start(1, 1)

        def body(t, carry):
            for j in range(4):
                k_ = 4 * t + j
                sl = j
                ld = (j + 2) % 4
                if j < 2:
                    @pl.when(t > 0)
                    def _(ld=ld):
                        s_wait(ld)
                else:
                    s_wait(ld)
                l_start(k_ + 2, ld)
                l_wait(k_, sl)
                s_start(sl)
            return carry

        lax.fori_loop(0, 19, body, 0)

        s_wait(2)
        l_wait(_FULL - 2, 0)
        s_start(0)
        s_wait(3)
        l_wait(_FULL - 1, 1)
        s_start(1)

        s0 = base + _FULL * _CH
        pltpu.sync_copy(dst_h.at[pl.ds(s0, _TAIL)], idxt_v)
        pltpu.sync_copy(ea_h.at[pl.ds(s0, _TAIL), :], valt_v)
        pltpu.sync_copy(valt_v, agg_sh.at[idxt_v], add=True)
        s_wait(0)
        s_wait(1)
        plsc.subcore_barrier()

        @pl.when(sid < _NCP)
        def _():
            pltpu.sync_copy(agg_sh.at[pl.ds(sid * _NPS, _NPS)],
                            out_h.at[cid, pl.ds(sid * _NPS, _NPS)])

    return k(ea, dstp)


def kernel(x, edge_index, edge_attr, u, batch, polar_pos,
           eW1, eb1, eW2, eb2, nW1, nb1, nW2, nb2, gW1, gb1, gW2, gb2):
    src = edge_index[0]
    dst = edge_index[1]
    batchf = batch.astype(_F32).reshape(_N, 1)

    eWs, eWd = eW1[:_D], eW1[_D:2 * _D]
    eWe, eWu = eW1[2 * _D:2 * _D + _DE], eW1[2 * _D + _DE:]
    nWx, nWa, nWu = nW1[:_D], nW1[_D:_D + _DE], nW1[_D + _DE:]
    gWu, gWx, gWp = gW1[:_DU], gW1[_DU:_DU + _D], gW1[_DU + _D:]
    eb1r, eb2r = eb1.reshape(1, _H), eb2.reshape(1, _DE)
    nb1r, nb2r = nb1.reshape(1, _H), nb2.reshape(1, _D)
    gb1r, gb2r = gb1.reshape(1, _H), gb2.reshape(1, _DU)

    cnt, psum = _stats(batchf, polar_pos)

    x_cat, g_cat = [x], [u]
    ea = edge_attr
    for _ in range(_STEPS):
        ps, pd = _prep(x, batchf, u, eWs, eWd, eWu, eb1r)
        hsum = _gather_sc(ps, pd, src, dst)
        ea = _edge(hsum, ea, eWe, eW2, eb2r)
        aggp = _scatter_sc(ea, dst)
        x, xsum = _node(x, aggp[0], aggp[1], batchf, u, nWx, nWa, nWu,
                        nb1r, nW2, nb2r)
        u = _glob(u, xsum, psum, cnt, gWu, gWx, gWp, gb1r, gW2, gb2r)
        x_cat.append(x)
        g_cat.append(u)
    return jnp.concatenate(x_cat, axis=1), jnp.concatenate(g_cat, axis=1)


# speedup vs baseline: 6.2374x; 1.5713x over previous
import functools

import jax
import jax.numpy as jnp
from jax import lax
from jax.experimental import pallas as pl
from jax.experimental.pallas import tpu as pltpu
from jax.experimental.pallas import tpu_sc as plsc

_N = 10000
_E = 320000
_D = 128
_DE = 16
_G = 16
_DU = 32
_H = 128
_STEPS = 2

_F32 = jnp.float32
_HIGH = lax.Precision.DEFAULT

_TILE_N = 1000
_TILE_E = 6400

_NC = 2
_NS = 16
_NW = _NC * _NS
_EPW = _E // _NW
_CH = 128
_NCH = -(-_EPW // _CH)
_LAST = _EPW - _CH
_FULL = _EPW // _CH
_TAIL = _EPW - _FULL * _CH
_NCP = 10
_NPS = _N // _NCP
_NCH2 = 81
_NT = 26


def _oh16(b_block):
    iota = lax.broadcasted_iota(jnp.int32, (1, _G), 1).astype(_F32)
    return (b_block == iota).astype(_F32)


def _stats_body(b_r, p_r, cnt_r, ps_r):
    i = pl.program_id(0)
    oh = _oh16(b_r[...])

    @pl.when(i == 0)
    def _():
        cnt_r[...] = jnp.zeros_like(cnt_r)
        ps_r[...] = jnp.zeros_like(ps_r)

    ones = jnp.ones((_TILE_N, _H), _F32)
    cnt_r[...] += lax.dot_general(oh, ones, (((0,), (0,)), ((), ())),
                                  precision=_HIGH)
    ps_r[...] += lax.dot_general(oh, p_r[...], (((0,), (0,)), ((), ())),
                                 precision=_HIGH)


def _stats(batchf, polar):
    return pl.pallas_call(
        _stats_body,
        grid=(_N // _TILE_N,),
        in_specs=[
            pl.BlockSpec((_TILE_N, 1), lambda i: (i, 0)),
            pl.BlockSpec((_TILE_N, 2), lambda i: (i, 0)),
        ],
        out_specs=[
            pl.BlockSpec((_G, _H), lambda i: (0, 0)),
            pl.BlockSpec((_G, 2), lambda i: (0, 0)),
        ],
        out_shape=[
            jax.ShapeDtypeStruct((_G, _H), _F32),
            jax.ShapeDtypeStruct((_G, 2), _F32),
        ],
    )(batchf, polar)


def _prep_body(x_r, b_r, u_r, ws_r, wd_r, wu_r, eb1_r, ps_r, pd_r):
    x = x_r[...]
    oh = _oh16(b_r[...])
    uw = jnp.dot(u_r[...], wu_r[...], precision=_HIGH)
    ps_r[...] = (jnp.dot(x, ws_r[...], precision=_HIGH)
                 + jnp.dot(oh, uw, precision=_HIGH) + eb1_r[...])
    pd_r[...] = jnp.dot(x, wd_r[...], precision=_HIGH)


def _prep(x, batchf, u, ws, wd, wu, eb1):
    return pl.pallas_call(
        _prep_body,
        grid=(_N // _TILE_N,),
        in_specs=[
            pl.BlockSpec((_TILE_N, _D), lambda i: (i, 0)),
            pl.BlockSpec((_TILE_N, 1), lambda i: (i, 0)),
            pl.BlockSpec((_G, _DU), lambda i: (0, 0)),
            pl.BlockSpec((_D, _H), lambda i: (0, 0)),
            pl.BlockSpec((_D, _H), lambda i: (0, 0)),
            pl.BlockSpec((_DU, _H), lambda i: (0, 0)),
            pl.BlockSpec((1, _H), lambda i: (0, 0)),
        ],
        out_specs=[pl.BlockSpec((_TILE_N, _H), lambda i: (i, 0))] * 2,
        out_shape=[jax.ShapeDtypeStruct((_N, _H), _F32)] * 2,
    )(x, batchf, u, ws, wd, wu, eb1)


def _edge_body(hsum_r, ea_r, we_r, w2_r, eb2_r, out_r):
    eaw = jnp.dot(ea_r[...], we_r[...], precision=_HIGH)
    h = jnp.maximum(hsum_r[...] + eaw, 0.0)
    out_r[...] = jnp.dot(h, w2_r[...], precision=_HIGH) + eb2_r[...]


def _edge(hsum, ea, we, w2, eb2r):
    return pl.pallas_call(
        _edge_body,
        grid=(_E // _TILE_E,),
        in_specs=[
            pl.BlockSpec((_TILE_E, _H), lambda i: (i, 0)),
            pl.BlockSpec((_TILE_E, _DE), lambda i: (i, 0)),
            pl.BlockSpec((_DE, _H), lambda i: (0, 0)),
            pl.BlockSpec((_H, _DE), lambda i: (0, 0)),
            pl.BlockSpec((1, _DE), lambda i: (0, 0)),
        ],
        out_specs=pl.BlockSpec((_TILE_E, _DE), lambda i: (i, 0)),
        out_shape=jax.ShapeDtypeStruct((_E, _DE), _F32),
    )(hsum, ea, we, w2, eb2r)


def _node_body(x_r, a0_r, a1_r, b_r, u_r, wx_r, wa_r, wu_r, nb1_r, w2_r,
               nb2_r, xn_r, xs_r):
    i = pl.program_id(0)
    x = x_r[...]
    oh = _oh16(b_r[...])
    uw = jnp.dot(u_r[...], wu_r[...], precision=_HIGH)
    agg = a0_r[...] + a1_r[...]
    npre = (jnp.dot(x, wx_r[...], precision=_HIGH)
            + jnp.dot(agg, wa_r[...], precision=_HIGH)
            + jnp.dot(oh, uw, precision=_HIGH) + nb1_r[...])
    xn = jnp.dot(jnp.maximum(npre, 0.0), w2_r[...], precision=_HIGH) + nb2_r[...]
    xn_r[...] = xn

    @pl.when(i == 0)
    def _():
        xs_r[...] = jnp.zeros_like(xs_r)

    xs_r[...] += lax.dot_general(oh, xn, (((0,), (0,)), ((), ())),
                                 precision=_HIGH)


def _node(x, a0, a1, batchf, u, wx, wa, wu, nb1, w2, nb2):
    return pl.pallas_call(
        _node_body,
        grid=(_N // _TILE_N,),
        in_specs=[
            pl.BlockSpec((_TILE_N, _D), lambda i: (i, 0)),
            pl.BlockSpec((_TILE_N, _DE), lambda i: (i, 0)),
            pl.BlockSpec((_TILE_N, _DE), lambda i: (i, 0)),
            pl.BlockSpec((_TILE_N, 1), lambda i: (i, 0)),
            pl.BlockSpec((_G, _DU), lambda i: (0, 0)),
            pl.BlockSpec((_D, _H), lambda i: (0, 0)),
            pl.BlockSpec((_DE, _H), lambda i: (0, 0)),
            pl.BlockSpec((_DU, _H), lambda i: (0, 0)),
            pl.BlockSpec((1, _H), lambda i: (0, 0)),
            pl.BlockSpec((_H, _D), lambda i: (0, 0)),
            pl.BlockSpec((1, _D), lambda i: (0, 0)),
        ],
        out_specs=[
            pl.BlockSpec((_TILE_N, _D), lambda i: (i, 0)),
            pl.BlockSpec((_G, _H), lambda i: (0, 0)),
        ],
        out_shape=[
            jax.ShapeDtypeStruct((_N, _D), _F32),
            jax.ShapeDtypeStruct((_G, _H), _F32),
        ],
    )(x, a0, a1, batchf, u, wx, wa, wu, nb1, w2, nb2)


def _glob_body(u_r, xs_r, ps_r, cnt_r, gwu_r, gwx_r, gwp_r, gb1_r, gw2_r,
               gb2_r, un_r):
    c = jnp.maximum(cnt_r[...], 1.0)
    xm = xs_r[...] / c
    pm = ps_r[...] / c[:, :2]
    gpre = (jnp.dot(u_r[...], gwu_r[...], precision=_HIGH)
            + jnp.dot(xm, gwx_r[...], precision=_HIGH)
            + jnp.dot(pm, gwp_r[...], precision=_HIGH) + gb1_r[...])
    un_r[...] = jnp.dot(jnp.maximum(gpre, 0.0), gw2_r[...],
                        precision=_HIGH) + gb2_r[...]


def _glob(u, xsum, psum, cnt, gwu, gwx, gwp, gb1, gw2, gb2):
    return pl.pallas_call(
        _glob_body,
        out_shape=jax.ShapeDtypeStruct((_G, _DU), _F32),
    )(u, xsum, psum, cnt, gwu, gwx, gwp, gb1, gw2, gb2)


def _gather_sc(ps, pd, srcp, dstp):
    mesh = plsc.VectorSubcoreMesh(core_axis_name="c", subcore_axis_name="s")

    @functools.partial(
        pl.kernel,
        mesh=mesh,
        out_type=jax.ShapeDtypeStruct((_E, _H), _F32),
        scratch_types=[
            pltpu.VMEM((_EPW,), jnp.int32),
            pltpu.VMEM((_EPW,), jnp.int32),
        ] + [pltpu.VMEM((_CH, _H), _F32)] * 6
          + [pltpu.SemaphoreType.DMA] * 9,
        compiler_params=pltpu.CompilerParams(use_tc_tiling_on_sc=False),
    )
    def k(ps_h, pd_h, src_h, dst_h, hsum_h, src_v, dst_v,
          rs0, rd0, rs1, rd1, rs2, rd2,
          gs0, gd0, w0, gs1, gd1, w1, gs2, gd2, w2):
        wid = lax.axis_index("s") * _NC + lax.axis_index("c")
        base = wid * _EPW
        pltpu.sync_copy(src_h.at[pl.ds(base, _EPW)], src_v)
        pltpu.sync_copy(dst_h.at[pl.ds(base, _EPW)], dst_v)

        rs = (rs0, rs1, rs2)
        rd = (rd0, rd1, rd2)
        gs = (gs0, gs1, gs2)
        gd = (gd0, gd1, gd2)
        ws = (w0, w1, w2)

        def s0_of(i):
            return jnp.minimum(i * _CH, _LAST)

        def g_start(i, sl):
            s0 = s0_of(i)
            pltpu.make_async_copy(
                ps_h.at[src_v.at[pl.ds(s0, _CH)]], rs[sl], gs[sl]).start()
            pltpu.make_async_copy(
                pd_h.at[dst_v.at[pl.ds(s0, _CH)]], rd[sl], gd[sl]).start()

        def g_wait(i, sl):
            s0 = s0_of(i)
            pltpu.make_async_copy(
                ps_h.at[src_v.at[pl.ds(s0, _CH)]], rs[sl], gs[sl]).wait()
            pltpu.make_async_copy(
                pd_h.at[dst_v.at[pl.ds(s0, _CH)]], rd[sl], gd[sl]).wait()

        def compute(sl):
            a, b = rs[sl], rd[sl]

            def cb(r, carry):
                for c in range(_H // 16):
                    s = pl.ds(c * 16, 16)
                    a[r, s] = a[r, s] + b[r, s]
                return carry

            lax.fori_loop(0, _CH, cb, 0)

        def w_start(i, sl):
            s0 = s0_of(i)
            pltpu.make_async_copy(
                rs[sl], hsum_h.at[pl.ds(base + s0, _CH)], ws[sl]).start()

        def w_wait(i, sl):
            s0 = s0_of(i)
            pltpu.make_async_copy(
                rs[sl], hsum_h.at[pl.ds(base + s0, _CH)], ws[sl]).wait()

        def chunk(i, carry):
            g_start(i, 0)
            g_wait(i, 0)
            compute(0)
            w_start(i, 0)
            w_wait(i, 0)
            return carry

        lax.fori_loop(0, _NCH, chunk, 0)

    return k(ps, pd, srcp, dstp)


def _scatter_sc(ea, dstp):
    mesh = plsc.VectorSubcoreMesh(core_axis_name="c", subcore_axis_name="s")

    @functools.partial(
        pl.kernel,
        mesh=mesh,
        out_type=jax.ShapeDtypeStruct((_NC, _N, _DE), _F32),
        scratch_types=(
            [pltpu.VMEM((_CH,), jnp.int32)] * 4
            + [pltpu.VMEM((_CH, _DE), _F32)] * 4
            + [
                pltpu.VMEM((_TAIL,), jnp.int32),
                pltpu.VMEM((_TAIL, _DE), _F32),
                pltpu.VMEM((_NPS, _DE), _F32),
                pltpu.VMEM_SHARED((_N, _DE), _F32),
            ]
            + [pltpu.SemaphoreType.DMA] * 8
        ),
        compiler_params=pltpu.CompilerParams(use_tc_tiling_on_sc=False),
    )
    def k(ea_h, dst_h, out_h, idx0, idx1, idx2, idx3,
          val0, val1, val2, val3, idxt_v, valt_v, z_v, agg_sh,
          cs0, cs1, cs2, cs3, ss0, ss1, ss2, ss3):
        cid = lax.axis_index("c")
        sid = lax.axis_index("s")
        wid = sid * _NC + cid
        base = wid * _EPW
        idxb = (idx0, idx1, idx2, idx3)
        val = (val0, val1, val2, val3)
        cs = (cs0, cs1, cs2, cs3)
        ss = (ss0, ss1, ss2, ss3)

        @pl.when(sid < _NCP)
        def _():
            def zb(i, carry):
                z_v[i, :] = jnp.zeros((_DE,), _F32)
                return carry

            lax.fori_loop(0, _NPS, zb, 0)
            pltpu.sync_copy(z_v, agg_sh.at[pl.ds(sid * _NPS, _NPS)])

        plsc.subcore_barrier()

        def l_start(i, sl):
            s0 = base + i * _CH
            pltpu.make_async_copy(
                dst_h.at[pl.ds(s0, _CH)], idxb[sl], cs[sl]).start()
            pltpu.make_async_copy(
                ea_h.at[pl.ds(s0, _CH), :], val[sl], cs[sl]).start()

        def l_wait(i, sl):
            s0 = base + i * _CH
            pltpu.make_async_copy(
                dst_h.at[pl.ds(s0, _CH)], idxb[sl], cs[sl]).wait()
            pltpu.make_async_copy(
                ea_h.at[pl.ds(s0, _CH), :], val[sl], cs[sl]).wait()

        def s_start(sl):
            pltpu.async_copy(val[sl], agg_sh.at[idxb[sl]], ss[sl], add=True)

        def s_wait(sl):
            pltpu.make_async_copy(val[sl], agg_sh.at[idxb[sl]], ss[sl]).wait()

        l_start(0, 0)
        l_start(1, 1)

        def body(t, carry):
            for j in range(4):
                k_ = 4 * t + j
                sl = j
                ld = (j + 2) % 4
                if j < 2:
                    @pl.when(t > 0)
                    def _(ld=ld):
                        s_wait(ld)
                else:
                    s_wait(ld)
                l_start(k_ + 2, ld)
                l_wait(k_, sl)
                s_start(sl)
            return carry

        lax.fori_loop(0, 19, body, 0)

        s_wait(2)
        l_wait(_FULL - 2, 0)
        s_start(0)
        s_wait(3)
        l_wait(_FULL - 1, 1)
        s_start(1)

        s0 = base + _FULL * _CH
        pltpu.sync_copy(dst_h.at[pl.ds(s0, _TAIL)], idxt_v)
        pltpu.sync_copy(ea_h.at[pl.ds(s0, _TAIL), :], valt_v)
        pltpu.sync_copy(valt_v, agg_sh.at[idxt_v], add=True)
        s_wait(0)
        s_wait(1)
        plsc.subcore_barrier()

        @pl.when(sid < _NCP)
        def _():
            pltpu.sync_copy(agg_sh.at[pl.ds(sid * _NPS, _NPS)],
                            out_h.at[cid, pl.ds(sid * _NPS, _NPS)])

    return k(ea, dstp)


def kernel(x, edge_index, edge_attr, u, batch, polar_pos,
           eW1, eb1, eW2, eb2, nW1, nb1, nW2, nb2, gW1, gb1, gW2, gb2):
    src = edge_index[0]
    dst = edge_index[1]
    batchf = batch.astype(_F32).reshape(_N, 1)

    eWs, eWd = eW1[:_D], eW1[_D:2 * _D]
    eWe, eWu = eW1[2 * _D:2 * _D + _DE], eW1[2 * _D + _DE:]
    nWx, nWa, nWu = nW1[:_D], nW1[_D:_D + _DE], nW1[_D + _DE:]
    gWu, gWx, gWp = gW1[:_DU], gW1[_DU:_DU + _D], gW1[_DU + _D:]
    eb1r, eb2r = eb1.reshape(1, _H), eb2.reshape(1, _DE)
    nb1r, nb2r = nb1.reshape(1, _H), nb2.reshape(1, _D)
    gb1r, gb2r = gb1.reshape(1, _H), gb2.reshape(1, _DU)

    cnt, psum = _stats(batchf, polar_pos)

    x_cat, g_cat = [x], [u]
    ea = edge_attr
    for _ in range(_STEPS):
        ps, pd = _prep(x, batchf, u, eWs, eWd, eWu, eb1r)
        hsum = _gather_sc(ps, pd, src, dst)
        ea = _edge(hsum, ea, eWe, eW2, eb2r)
        aggp = _scatter_sc(ea, dst)
        x, xsum = _node(x, aggp[0], aggp[1], batchf, u, nWx, nWa, nWu,
                        nb1r, nW2, nb2r)
        u = _glob(u, xsum, psum, cnt, gWu, gWx, gWp, gb1r, gW2, gb2r)
        x_cat.append(x)
        g_cat.append(u)
    return jnp.concatenate(x_cat, axis=1), jnp.concatenate(g_cat, axis=1)


# speedup vs baseline: 6.2390x; 1.0003x over previous
import functools

import jax
import jax.numpy as jnp
from jax import lax
from jax.experimental import pallas as pl
from jax.experimental.pallas import tpu as pltpu
from jax.experimental.pallas import tpu_sc as plsc

_N = 10000
_E = 320000
_D = 128
_DE = 16
_G = 16
_DU = 32
_H = 128
_STEPS = 2

_F32 = jnp.float32
_HIGH = lax.Precision.DEFAULT

_TILE_N = 1000
_TILE_E = 6400

_NC = 2
_NS = 16
_NW = _NC * _NS
_EPW = _E // _NW
_CH = 128
_NCH = -(-_EPW // _CH)
_LAST = _EPW - _CH
_FULL = _EPW // _CH
_TAIL = _EPW - _FULL * _CH
_NCP = 10
_NPS = _N // _NCP
_NCH2 = 81
_NT = 26


def _oh16(b_block):
    iota = lax.broadcasted_iota(jnp.int32, (1, _G), 1).astype(_F32)
    return (b_block == iota).astype(_F32)


def _stats_body(b_r, p_r, cnt_r, ps_r):
    i = pl.program_id(0)
    oh = _oh16(b_r[...])

    @pl.when(i == 0)
    def _():
        cnt_r[...] = jnp.zeros_like(cnt_r)
        ps_r[...] = jnp.zeros_like(ps_r)

    ones = jnp.ones((_TILE_N, _H), _F32)
    cnt_r[...] += lax.dot_general(oh, ones, (((0,), (0,)), ((), ())),
                                  precision=_HIGH)
    ps_r[...] += lax.dot_general(oh, p_r[...], (((0,), (0,)), ((), ())),
                                 precision=_HIGH)


def _stats(batchf, polar):
    return pl.pallas_call(
        _stats_body,
        grid=(_N // _TILE_N,),
        in_specs=[
            pl.BlockSpec((_TILE_N, 1), lambda i: (i, 0)),
            pl.BlockSpec((_TILE_N, 2), lambda i: (i, 0)),
        ],
        out_specs=[
            pl.BlockSpec((_G, _H), lambda i: (0, 0)),
            pl.BlockSpec((_G, 2), lambda i: (0, 0)),
        ],
        out_shape=[
            jax.ShapeDtypeStruct((_G, _H), _F32),
            jax.ShapeDtypeStruct((_G, 2), _F32),
        ],
    )(batchf, polar)


def _prep_body(x_r, b_r, u_r, ws_r, wd_r, wu_r, eb1_r, ps_r, pd_r):
    x = x_r[...]
    oh = _oh16(b_r[...])
    uw = jnp.dot(u_r[...], wu_r[...], precision=_HIGH)
    ps_r[...] = (jnp.dot(x, ws_r[...], precision=_HIGH)
                 + jnp.dot(oh, uw, precision=_HIGH) + eb1_r[...])
    pd_r[...] = jnp.dot(x, wd_r[...], precision=_HIGH)


def _prep(x, batchf, u, ws, wd, wu, eb1):
    return pl.pallas_call(
        _prep_body,
        grid=(_N // _TILE_N,),
        in_specs=[
            pl.BlockSpec((_TILE_N, _D), lambda i: (i, 0)),
            pl.BlockSpec((_TILE_N, 1), lambda i: (i, 0)),
            pl.BlockSpec((_G, _DU), lambda i: (0, 0)),
            pl.BlockSpec((_D, _H), lambda i: (0, 0)),
            pl.BlockSpec((_D, _H), lambda i: (0, 0)),
            pl.BlockSpec((_DU, _H), lambda i: (0, 0)),
            pl.BlockSpec((1, _H), lambda i: (0, 0)),
        ],
        out_specs=[pl.BlockSpec((_TILE_N, _H), lambda i: (i, 0))] * 2,
        out_shape=[jax.ShapeDtypeStruct((_N, _H), _F32)] * 2,
    )(x, batchf, u, ws, wd, wu, eb1)


def _edge_body(hsum_r, ea_r, we_r, w2_r, eb2_r, out_r):
    eaw = jnp.dot(ea_r[...], we_r[...], precision=_HIGH)
    h = jnp.maximum(hsum_r[...] + eaw, 0.0)
    out_r[...] = jnp.dot(h, w2_r[...], precision=_HIGH) + eb2_r[...]


def _edge(hsum, ea, we, w2, eb2r):
    return pl.pallas_call(
        _edge_body,
        grid=(_E // _TILE_E,),
        in_specs=[
            pl.BlockSpec((_TILE_E, _H), lambda i: (i, 0)),
            pl.BlockSpec((_TILE_E, _DE), lambda i: (i, 0)),
            pl.BlockSpec((_DE, _H), lambda i: (0, 0)),
            pl.BlockSpec((_H, _DE), lambda i: (0, 0)),
            pl.BlockSpec((1, _DE), lambda i: (0, 0)),
        ],
        out_specs=pl.BlockSpec((_TILE_E, _DE), lambda i: (i, 0)),
        out_shape=jax.ShapeDtypeStruct((_E, _DE), _F32),
    )(hsum, ea, we, w2, eb2r)


def _node_body(x_r, a0_r, a1_r, b_r, u_r, wx_r, wa_r, wu_r,
               nb1_r, w2_r, nb2_r, xn_r, xs_r):
    i = pl.program_id(0)
    x = x_r[...]
    oh = _oh16(b_r[...])
    uw = jnp.dot(u_r[...], wu_r[...], precision=_HIGH)
    agg = a0_r[...] + a1_r[...]
    npre = (jnp.dot(x, wx_r[...], precision=_HIGH)
            + jnp.dot(agg, wa_r[...], precision=_HIGH)
            + jnp.dot(oh, uw, precision=_HIGH) + nb1_r[...])
    xn = jnp.dot(jnp.maximum(npre, 0.0), w2_r[...], precision=_HIGH) + nb2_r[...]
    xn_r[...] = xn

    @pl.when(i == 0)
    def _():
        xs_r[...] = jnp.zeros_like(xs_r)

    xs_r[...] += lax.dot_general(oh, xn, (((0,), (0,)), ((), ())),
                                 precision=_HIGH)


def _node(x, a0, a1, batchf, u, wx, wa, wu, nb1, w2, nb2):
    return pl.pallas_call(
        _node_body,
        grid=(_N // _TILE_N,),
        in_specs=[
            pl.BlockSpec((_TILE_N, _D), lambda i: (i, 0)),
            pl.BlockSpec((_TILE_N, _DE), lambda i: (i, 0)),
            pl.BlockSpec((_TILE_N, _DE), lambda i: (i, 0)),
            pl.BlockSpec((_TILE_N, 1), lambda i: (i, 0)),
            pl.BlockSpec((_G, _DU), lambda i: (0, 0)),
            pl.BlockSpec((_D, _H), lambda i: (0, 0)),
            pl.BlockSpec((_DE, _H), lambda i: (0, 0)),
            pl.BlockSpec((_DU, _H), lambda i: (0, 0)),
            pl.BlockSpec((1, _H), lambda i: (0, 0)),
            pl.BlockSpec((_H, _D), lambda i: (0, 0)),
            pl.BlockSpec((1, _D), lambda i: (0, 0)),
        ],
        out_specs=[
            pl.BlockSpec((_TILE_N, _D), lambda i: (i, 0)),
            pl.BlockSpec((_G, _H), lambda i: (0, 0)),
        ],
        out_shape=[
            jax.ShapeDtypeStruct((_N, _D), _F32),
            jax.ShapeDtypeStruct((_G, _H), _F32),
        ],
    )(x, a0, a1, batchf, u, wx, wa, wu, nb1, w2, nb2)


def _glob_body(u_r, xs_r, ps_r, cnt_r, gwu_r, gwx_r, gwp_r, gb1_r, gw2_r,
               gb2_r, un_r):
    c = jnp.maximum(cnt_r[...], 1.0)
    xm = xs_r[...] / c
    pm = ps_r[...] / c[:, :2]
    gpre = (jnp.dot(u_r[...], gwu_r[...], precision=_HIGH)
            + jnp.dot(xm, gwx_r[...], precision=_HIGH)
            + jnp.dot(pm, gwp_r[...], precision=_HIGH) + gb1_r[...])
    un_r[...] = jnp.dot(jnp.maximum(gpre, 0.0), gw2_r[...],
                        precision=_HIGH) + gb2_r[...]


def _glob(u, xsum, psum, cnt, gwu, gwx, gwp, gb1, gw2, gb2):
    return pl.pallas_call(
        _glob_body,
        out_shape=jax.ShapeDtypeStruct((_G, _DU), _F32),
    )(u, xsum, psum, cnt, gwu, gwx, gwp, gb1, gw2, gb2)


def _gather_sc(ps, pd, srcp, dstp):
    es = srcp.shape[0]
    epw = es // _NW
    nch = -(-epw // _CH)
    last = epw - _CH
    mesh = plsc.VectorSubcoreMesh(core_axis_name="c", subcore_axis_name="s")

    @functools.partial(
        pl.kernel,
        mesh=mesh,
        out_type=jax.ShapeDtypeStruct((es, _H), _F32),
        scratch_types=[
            pltpu.VMEM((epw,), jnp.int32),
            pltpu.VMEM((epw,), jnp.int32),
        ] + [pltpu.VMEM((_CH, _H), _F32)] * 6
          + [pltpu.SemaphoreType.DMA] * 9,
        compiler_params=pltpu.CompilerParams(use_tc_tiling_on_sc=False),
    )
    def k(ps_h, pd_h, src_h, dst_h, hsum_h, src_v, dst_v,
          rs0, rd0, rs1, rd1, rs2, rd2,
          gs0, gd0, w0, gs1, gd1, w1, gs2, gd2, w2):
        wid = lax.axis_index("s") * _NC + lax.axis_index("c")
        base = wid * epw
        pltpu.sync_copy(src_h.at[pl.ds(base, epw)], src_v)
        pltpu.sync_copy(dst_h.at[pl.ds(base, epw)], dst_v)

        rs = (rs0, rs1, rs2)
        rd = (rd0, rd1, rd2)
        gs = (gs0, gs1, gs2)
        gd = (gd0, gd1, gd2)
        ws = (w0, w1, w2)

        def s0_of(i):
            return jnp.minimum(i * _CH, last)

        def g_start(i, sl):
            s0 = s0_of(i)
            pltpu.make_async_copy(
                ps_h.at[src_v.at[pl.ds(s0, _CH)]], rs[sl], gs[sl]).start()
            pltpu.make_async_copy(
                pd_h.at[dst_v.at[pl.ds(s0, _CH)]], rd[sl], gd[sl]).start()

        def g_wait(i, sl):
            s0 = s0_of(i)
            pltpu.make_async_copy(
                ps_h.at[src_v.at[pl.ds(s0, _CH)]], rs[sl], gs[sl]).wait()
            pltpu.make_async_copy(
                pd_h.at[dst_v.at[pl.ds(s0, _CH)]], rd[sl], gd[sl]).wait()

        def compute(sl):
            a, b = rs[sl], rd[sl]

            def cb(r, carry):
                for c in range(_H // 16):
                    s = pl.ds(c * 16, 16)
                    a[r, s] = a[r, s] + b[r, s]
                return carry

            lax.fori_loop(0, _CH, cb, 0)

        def w_start(i, sl):
            s0 = s0_of(i)
            pltpu.make_async_copy(
                rs[sl], hsum_h.at[pl.ds(base + s0, _CH)], ws[sl]).start()

        def w_wait(i, sl):
            s0 = s0_of(i)
            pltpu.make_async_copy(
                rs[sl], hsum_h.at[pl.ds(base + s0, _CH)], ws[sl]).wait()

        def chunk(i, carry):
            g_start(i, 0)
            g_wait(i, 0)
            compute(0)
            w_start(i, 0)
            w_wait(i, 0)
            return carry

        lax.fori_loop(0, nch, chunk, 0)

    return k(ps, pd, srcp, dstp)


def _scatter_sc(ea, dstp):
    es = dstp.shape[0]
    epw = es // _NW
    full = epw // _CH
    tail = epw - full * _CH
    trips = (full - 2) // 4
    mesh = plsc.VectorSubcoreMesh(core_axis_name="c", subcore_axis_name="s")

    @functools.partial(
        pl.kernel,
        mesh=mesh,
        out_type=jax.ShapeDtypeStruct((_NC, _N, _DE), _F32),
        scratch_types=(
            [pltpu.VMEM((_CH,), jnp.int32)] * 4
            + [pltpu.VMEM((_CH, _DE), _F32)] * 4
            + [
                pltpu.VMEM((tail, ), jnp.int32),
                pltpu.VMEM((tail, _DE), _F32),
                pltpu.VMEM((_NPS, _DE), _F32),
                pltpu.VMEM_SHARED((_N, _DE), _F32),
            ]
            + [pltpu.SemaphoreType.DMA] * 8
        ),
        compiler_params=pltpu.CompilerParams(use_tc_tiling_on_sc=False),
    )
    def k(ea_h, dst_h, out_h, idx0, idx1, idx2, idx3,
          val0, val1, val2, val3, idxt_v, valt_v, z_v, agg_sh,
          cs0, cs1, cs2, cs3, ss0, ss1, ss2, ss3):
        cid = lax.axis_index("c")
        sid = lax.axis_index("s")
        wid = sid * _NC + cid
        base = wid * epw
        idxb = (idx0, idx1, idx2, idx3)
        val = (val0, val1, val2, val3)
        cs = (cs0, cs1, cs2, cs3)
        ss = (ss0, ss1, ss2, ss3)

        @pl.when(sid < _NCP)
        def _():
            def zb(i, carry):
                z_v[i, :] = jnp.zeros((_DE,), _F32)
                return carry

            lax.fori_loop(0, _NPS, zb, 0)
            pltpu.sync_copy(z_v, agg_sh.at[pl.ds(sid * _NPS, _NPS)])

        plsc.subcore_barrier()

        def l_start(i, sl):
            s0 = base + i * _CH
            pltpu.make_async_copy(
                dst_h.at[pl.ds(s0, _CH)], idxb[sl], cs[sl]).start()
            pltpu.make_async_copy(
                ea_h.at[pl.ds(s0, _CH), :], val[sl], cs[sl]).start()

        def l_wait(i, sl):
            s0 = base + i * _CH
            pltpu.make_async_copy(
                dst_h.at[pl.ds(s0, _CH)], idxb[sl], cs[sl]).wait()
            pltpu.make_async_copy(
                ea_h.at[pl.ds(s0, _CH), :], val[sl], cs[sl]).wait()

        def s_start(sl):
            pltpu.async_copy(val[sl], agg_sh.at[idxb[sl]], ss[sl], add=True)

        def s_wait(sl):
            pltpu.make_async_copy(val[sl], agg_sh.at[idxb[sl]], ss[sl]).wait()

        l_start(0, 0)
        l_start(1, 1)

        def body(t, carry):
            for j in range(4):
                k_ = 4 * t + j
                sl = j
                ld = (j + 2) % 4
                if j < 2:
                    @pl.when(t > 0)
                    def _(ld=ld):
                        s_wait(ld)
                else:
                    s_wait(ld)
                l_start(k_ + 2, ld)
                l_wait(k_, sl)
                s_start(sl)
            return carry

        lax.fori_loop(0, trips, body, 0)

        s_wait(2)
        s_wait(3)
        for c in range(4 * trips, full):
            sl = c % 4
            if sl >= 2:
                l_start(c, sl)
            l_wait(c, sl)
            s_start(sl)

        s0 = base + full * _CH
        pltpu.sync_copy(dst_h.at[pl.ds(s0, tail)], idxt_v)
        pltpu.sync_copy(ea_h.at[pl.ds(s0, tail), :], valt_v)
        pltpu.sync_copy(valt_v, agg_sh.at[idxt_v], add=True)
        for c in range(4 * trips, full):
            s_wait(c % 4)
        plsc.subcore_barrier()

        @pl.when(sid < _NCP)
        def _():
            pltpu.sync_copy(agg_sh.at[pl.ds(sid * _NPS, _NPS)],
                            out_h.at[cid, pl.ds(sid * _NPS, _NPS)])

    return k(ea, dstp)


def kernel(x, edge_index, edge_attr, u, batch, polar_pos,
           eW1, eb1, eW2, eb2, nW1, nb1, nW2, nb2, gW1, gb1, gW2, gb2):
    src = edge_index[0]
    dst = edge_index[1]
    batchf = batch.astype(_F32).reshape(_N, 1)

    eWs, eWd = eW1[:_D], eW1[_D:2 * _D]
    eWe, eWu = eW1[2 * _D:2 * _D + _DE], eW1[2 * _D + _DE:]
    nWx, nWa, nWu = nW1[:_D], nW1[_D:_D + _DE], nW1[_D + _DE:]
    gWu, gWx, gWp = gW1[:_DU], gW1[_DU:_DU + _D], gW1[_DU + _D:]
    eb1r, eb2r = eb1.reshape(1, _H), eb2.reshape(1, _DE)
    nb1r, nb2r = nb1.reshape(1, _H), nb2.reshape(1, _D)
    gb1r, gb2r = gb1.reshape(1, _H), gb2.reshape(1, _DU)

    cnt, psum = _stats(batchf, polar_pos)

    x_cat, g_cat = [x], [u]
    ea = edge_attr
    for _ in range(_STEPS):
        ps, pd = _prep(x, batchf, u, eWs, eWd, eWu, eb1r)
        hsum = _gather_sc(ps, pd, src, dst)
        ea = _edge(hsum, ea, eWe, eW2, eb2r)
        aggp = _scatter_sc(ea, dst)
        x, xsum = _node(x, aggp[0], aggp[1], batchf, u, nWx, nWa, nWu,
                        nb1r, nW2, nb2r)
        u = _glob(u, xsum, psum, cnt, gWu, gWx, gWp, gb1r, gW2, gb2r)
        x_cat.append(x)
        g_cat.append(u)
    return jnp.concatenate(x_cat, axis=1), jnp.concatenate(g_cat, axis=1)
